# Initial kernel scaffold; baseline (speedup 1.0000x reference)
import jax
import jax.numpy as jnp
from jax.experimental import pallas as pl


def kernel(x, W_enc, b_enc, W_dec, b_dec):
    raise NotImplementedError("write your pallas kernel here")


# speedup vs baseline: 5.7931x; 5.7931x over previous
import functools

import jax
import jax.numpy as jnp
from jax import lax
from jax.experimental import pallas as pl
from jax.experimental.pallas import tpu as pltpu
from jax.experimental.pallas import tpu_sc as plsc

D_IN = 2048
D_SAE = 32768
KTOP = 64
B = 2048


BL = 1024
NLB = D_SAE // BL
NCHUNK = D_SAE // 8


def _encode_body(x_ref, w_ref, be_ref, bd_ref, pre_ref, cm_ref):
    xc = x_ref[...] - bd_ref[...]
    acc = lax.dot_general(xc, w_ref[...], (((1,), (1,)), ((), ())),
                          preferred_element_type=jnp.float32)
    pre = acc + be_ref[...]
    pre_ref[...] = pre
    cm = pre[:, 0:128]
    for m in range(1, 8):
        cm = jnp.maximum(cm, pre[:, 128 * m:128 * (m + 1)])
    cm_ref[...] = cm


def _encode(x, w_enc, b_enc, b_dec):
    return pl.pallas_call(
        _encode_body,
        grid=(2, NLB),
        in_specs=[
            pl.BlockSpec((B // 2, D_IN), lambda r, l: (r, 0)),
            pl.BlockSpec((BL, D_IN), lambda r, l: (l, 0)),
            pl.BlockSpec((1, BL), lambda r, l: (0, l)),
            pl.BlockSpec((1, D_IN), lambda r, l: (0, 0)),
        ],
        out_specs=[
            pl.BlockSpec((B // 2, BL), lambda r, l: (r, l)),
            pl.BlockSpec((B // 2, 128), lambda r, l: (r, l)),
        ],
        out_shape=[
            jax.ShapeDtypeStruct((B, D_SAE), jnp.float32),
            jax.ShapeDtypeStruct((B, NCHUNK), jnp.float32),
        ],
        compiler_params=pltpu.CompilerParams(
            dimension_semantics=("arbitrary", "arbitrary")),
    )(x, w_enc, b_enc.reshape(1, D_SAE), b_dec.reshape(1, D_IN))


NC = 2
NS = 16
NW = NC * NS
RPW = B // NW
CAP = 128
NEL = CAP * 8

_I32 = jnp.int32


def _mono(v):
    b = lax.bitcast_convert_type(v, _I32)
    return b ^ (jnp.right_shift(b, 31) & _I32(0x7FFFFFFF))


def _vext(vec, lane):
    return jnp.sum(jnp.where(lax.iota(_I32, 16) == lane, vec,
                             jnp.zeros((16,), vec.dtype)))


def _zero_hist(hist):
    z16 = jnp.zeros((16,), _I32)
    for i in range(16):
        hist[pl.ds(i * 16, 16)] = z16


def _scan_hist(hist, target):
    def ph1(i, c):
        acc, cross, accb = c
        t = jnp.sum(hist[pl.ds(i * 16, 16)])
        hit = jnp.logical_and(acc + t > target, cross < 0)
        cross = jnp.where(hit, i, cross)
        accb = jnp.where(hit, acc, accb)
        return acc + t, cross, accb

    _, cross, accb = lax.fori_loop(0, 16, ph1, (_I32(0), _I32(-1), _I32(0)))
    h = hist[pl.ds(cross * 16, 16)]
    cs = plsc.cumsum(h)
    m = (accb + cs) > target
    lane = jnp.max(plsc.all_reduce_ffs(m))
    b = cross * 16 + lane
    cb = accb + _vext(cs, lane)
    hb = _vext(h, lane)
    return b, cb, hb


def _histogram_pass(hist, n_vregs, get_bucket_mask):
    def body(t, carry):
        bkt, msk = get_bucket_mask(t)
        cnt, last = plsc.scan_count(bkt, mask=msk)
        plsc.addupdate_scatter(hist, [bkt], cnt, mask=last)
        return carry

    lax.fori_loop(0, n_vregs, body, 0)


def _sc_topk_kernel(pre_hbm, cmax_hbm, z_hbm, vals_hbm, idx_hbm,
                    cm_v, s_chunk, hist, cand_ch, eidx, colidx, s_elem,
                    cand_v, vals_buf, idx_buf, z_buf,
                    sem_g0, sem_g1, sem_z0, sem_z1):
    cid = lax.axis_index("c")
    sid = lax.axis_index("s")
    wid = sid * NC + cid
    row0 = wid * RPW
    iota = lax.iota(_I32, 16)
    zero16f = jnp.zeros((16,), jnp.float32)

    def zb(i, c):
        z_buf[pl.ds(i * 16, 16)] = zero16f
        return c
    lax.fori_loop(0, 2 * D_SAE // 16, zb, 0)

    def chunk_stage(r, gbuf):
        pltpu.sync_copy(cmax_hbm.at[pl.ds(r * NCHUNK, NCHUNK)], cm_v)

        _zero_hist(hist)

        def l1(t):
            s = _mono(cm_v[pl.ds(t * 16, 16)])
            s_chunk[pl.ds(t * 16, 16)] = s
            b = jnp.right_shift(s, 24) + 128
            return b, None
        _histogram_pass(hist, NCHUNK // 16, l1)
        b1, c1, h1 = _scan_hist(hist, _I32(NCHUNK - KTOP))
        a1 = NCHUNK - c1

        _zero_hist(hist)

        def l2(t):
            s = s_chunk[pl.ds(t * 16, 16)]
            msk = (jnp.right_shift(s, 24) + 128) == b1
            b = jnp.right_shift(s, 16) & 0xFF
            return b, msk
        _histogram_pass(hist, NCHUNK // 16, l2)
        b2, _, _ = _scan_hist(hist, h1 - (KTOP - a1))
        floor16 = jnp.left_shift(b1 - 128, 24) | jnp.left_shift(b2, 16)

        for j in range(CAP // 16):
            cand_ch[pl.ds(j * 16, 16)] = iota + j * 16

        def sel(t, n):
            s = s_chunk[pl.ds(t * 16, 16)]
            m = s >= floor16
            pc = plsc.cumsum(m.astype(_I32))
            keep = jnp.logical_and(m, (n + pc) <= CAP)
            plsc.store_compressed(cand_ch.at[pl.ds(n, 16)],
                                  iota + t * 16, mask=keep)
            return n + jnp.sum(keep.astype(_I32))
        n_ch = lax.fori_loop(0, NCHUNK // 16, sel, _I32(0))

        rbase = r * D_SAE
        for j in range(CAP // 16):
            c = cand_ch[pl.ds(j * 16, 16)]
            base = jnp.left_shift(jnp.right_shift(c, 7), 10) + (c & 127)
            for m in range(8):
                t = j * 8 + m
                col = base + 128 * m
                colidx[pl.ds(gbuf * NEL + t * 16, 16)] = col
                eidx[pl.ds(gbuf * NEL + t * 16, 16)] = col + rbase
        return n_ch

    def elem_stage(r, n_ch, gbuf, p, sem_z):
        n_el = n_ch * 8
        eoff = gbuf * NEL

        def valid_of(t):
            j = t // 8 if isinstance(t, int) else jnp.right_shift(t, 3)
            return (j * 16 + iota) < n_ch

        _zero_hist(hist)

        def l1(t):
            v = cand_v[pl.ds(eoff + t * 16, 16)]
            s = _mono(v)
            s_elem[pl.ds(eoff + t * 16, 16)] = s
            return jnp.right_shift(s, 24) + 128, valid_of(t)
        _histogram_pass(hist, NEL // 16, l1)
        b1, c1, h1 = _scan_hist(hist, n_el - KTOP)
        a1 = n_el - c1

        _zero_hist(hist)

        def l2(t):
            s = s_elem[pl.ds(eoff + t * 16, 16)]
            msk = jnp.logical_and(valid_of(t),
                                  (jnp.right_shift(s, 24) + 128) == b1)
            return jnp.right_shift(s, 16) & 0xFF, msk
        _histogram_pass(hist, NEL // 16, l2)
        b2, c2, h2 = _scan_hist(hist, h1 - (KTOP - a1))
        a2 = h1 - c2
        pre16 = jnp.left_shift(b1 - 128, 8) | b2

        _zero_hist(hist)

        def l3(t):
            s = s_elem[pl.ds(eoff + t * 16, 16)]
            msk = jnp.logical_and(valid_of(t),
                                  jnp.right_shift(s, 16) == pre16)
            return jnp.right_shift(s, 8) & 0xFF, msk
        _histogram_pass(hist, NEL // 16, l3)
        b3, c3, h3 = _scan_hist(hist, h2 - (KTOP - a1 - a2))
        a3 = h2 - c3
        pre24 = jnp.left_shift(pre16, 8) | b3

        _zero_hist(hist)

        def l4(t):
            s = s_elem[pl.ds(eoff + t * 16, 16)]
            msk = jnp.logical_and(valid_of(t),
                                  jnp.right_shift(s, 8) == pre24)
            return s & 0xFF, msk
        _histogram_pass(hist, NEL // 16, l4)
        b4, _, _ = _scan_hist(hist, h3 - (KTOP - a1 - a2 - a3))
        s_star = jnp.left_shift(pre24, 8) | b4

        @pl.when(r - row0 >= 2)
        def _():
            pltpu.make_async_copy(z_buf.at[pl.ds(p * D_SAE, D_SAE)],
                                  z_hbm.at[pl.ds((r - 2) * D_SAE, D_SAE)],
                                  sem_z).wait()
            for j in range(KTOP // 16):
                ii = idx_buf[pl.ds(p * 80 + j * 16, 16)]
                plsc.store_scatter(z_buf, [ii + p * D_SAE], zero16f)

        def sel(t, n):
            s = s_elem[pl.ds(eoff + t * 16, 16)]
            m = jnp.logical_and(valid_of(t), s >= s_star)
            pc = plsc.cumsum(m.astype(_I32))
            keep = jnp.logical_and(m, (n + pc) <= KTOP)
            v = cand_v[pl.ds(eoff + t * 16, 16)]
            plsc.store_compressed(vals_buf.at[pl.ds(n, 16)],
                                  jnp.maximum(v, 0.0), mask=keep)
            plsc.store_compressed(idx_buf.at[pl.ds(p * 80 + n, 16)],
                                  colidx[pl.ds(eoff + t * 16, 16)],
                                  mask=keep)
            return n + jnp.sum(keep.astype(_I32))
        lax.fori_loop(0, NEL // 16, sel, _I32(0))

        for j in range(KTOP // 16):
            ii = idx_buf[pl.ds(p * 80 + j * 16, 16)]
            vv = vals_buf[pl.ds(j * 16, 16)]
            plsc.store_scatter(z_buf, [ii + p * D_SAE], vv)

        pltpu.async_copy(z_buf.at[pl.ds(p * D_SAE, D_SAE)],
                         z_hbm.at[pl.ds(r * D_SAE, D_SAE)], sem_z)
        pltpu.sync_copy(vals_buf.at[pl.ds(0, KTOP)],
                        vals_hbm.at[pl.ds(r * KTOP, KTOP)])
        pltpu.sync_copy(idx_buf.at[pl.ds(p * 80, KTOP)],
                        idx_hbm.at[pl.ds(r * KTOP, KTOP)])

    def pair(rr, c):
        r0 = row0 + 2 * rr
        r1 = r0 + 1
        n0 = chunk_stage(r0, 0)
        cpy0 = pltpu.async_copy(pre_hbm.at[eidx.at[pl.ds(0, NEL)]],
                                cand_v.at[pl.ds(0, NEL)], sem_g0)
        n1 = chunk_stage(r1, 1)
        cpy1 = pltpu.async_copy(pre_hbm.at[eidx.at[pl.ds(NEL, NEL)]],
                                cand_v.at[pl.ds(NEL, NEL)], sem_g1)
        cpy0.wait()
        elem_stage(r0, n0, 0, 0, sem_z0)
        cpy1.wait()
        elem_stage(r1, n1, 1, 1, sem_z1)
        return c

    lax.fori_loop(0, RPW // 2, pair, 0)

    pltpu.make_async_copy(z_buf.at[pl.ds(0, D_SAE)],
                          z_hbm.at[pl.ds((row0 + RPW - 2) * D_SAE, D_SAE)],
                          sem_z0).wait()
    pltpu.make_async_copy(z_buf.at[pl.ds(D_SAE, D_SAE)],
                          z_hbm.at[pl.ds((row0 + RPW - 1) * D_SAE, D_SAE)],
                          sem_z1).wait()


def _sc_topk(pre, cmax):
    mesh = plsc.VectorSubcoreMesh(core_axis_name="c", subcore_axis_name="s",
                                  num_cores=NC, num_subcores=NS)
    kfn = pl.kernel(
        _sc_topk_kernel,
        out_type=[
            jax.ShapeDtypeStruct((B * D_SAE,), jnp.float32),
            jax.ShapeDtypeStruct((B * KTOP,), jnp.float32),
            jax.ShapeDtypeStruct((B * KTOP,), jnp.int32),
        ],
        mesh=mesh,
        compiler_params=pltpu.CompilerParams(needs_layout_passes=False),
        scratch_types=[
            pltpu.VMEM((NCHUNK,), jnp.float32),
            pltpu.VMEM((NCHUNK,), jnp.int32),
            pltpu.VMEM((256,), jnp.int32),
            pltpu.VMEM((CAP + 16,), jnp.int32),
            pltpu.VMEM((2 * NEL,), jnp.int32),
            pltpu.VMEM((2 * NEL,), jnp.int32),
            pltpu.VMEM((2 * NEL,), jnp.int32),
            pltpu.VMEM((2 * NEL,), jnp.float32),
            pltpu.VMEM((80,), jnp.float32),
            pltpu.VMEM((160,), jnp.int32),
            pltpu.VMEM((2 * D_SAE,), jnp.float32),
            pltpu.SemaphoreType.DMA,
            pltpu.SemaphoreType.DMA,
            pltpu.SemaphoreType.DMA,
            pltpu.SemaphoreType.DMA,
        ],
    )
    zf, vf, inf = kfn(pre.reshape(-1), cmax.reshape(-1))
    return (zf.reshape(B, D_SAE), vf.reshape(B, KTOP),
            inf.reshape(B, KTOP))


KB = 1024
NKB = D_SAE // KB


def _decode_body(z_ref, w_ref, bd_ref, out_ref):
    kb = pl.program_id(0)

    @pl.when(kb == 0)
    def _():
        out_ref[...] = jnp.broadcast_to(bd_ref[...], (B, D_IN))

    out_ref[...] += lax.dot_general(
        z_ref[...], w_ref[...], (((1,), (1,)), ((), ())),
        preferred_element_type=jnp.float32)


def _decode(z, w_dec, b_dec):
    return pl.pallas_call(
        _decode_body,
        grid=(NKB,),
        in_specs=[
            pl.BlockSpec((B, KB), lambda k: (0, k)),
            pl.BlockSpec((D_IN, KB), lambda k: (0, k)),
            pl.BlockSpec((1, D_IN), lambda k: (0, 0)),
        ],
        out_specs=pl.BlockSpec((B, D_IN), lambda k: (0, 0)),
        out_shape=jax.ShapeDtypeStruct((B, D_IN), jnp.float32),
        compiler_params=pltpu.CompilerParams(
            dimension_semantics=("arbitrary",)),
    )(z, w_dec, b_dec.reshape(1, D_IN))


LRB = 256
NLRB = B // LRB


def _loss_body(xh_ref, x_ref, out_ref):
    rb = pl.program_id(0)
    d = xh_ref[...] - x_ref[...]
    s = jnp.sum(d * d).reshape(1, 1)

    @pl.when(rb == 0)
    def _():
        out_ref[...] = jnp.zeros((1, 1), jnp.float32)

    out_ref[...] += s

    @pl.when(rb == NLRB - 1)
    def _():
        out_ref[...] = out_ref[...] / B


def _loss(x_hat, x):
    return pl.pallas_call(
        _loss_body,
        grid=(NLRB,),
        in_specs=[
            pl.BlockSpec((LRB, D_IN), lambda r: (r, 0)),
            pl.BlockSpec((LRB, D_IN), lambda r: (r, 0)),
        ],
        out_specs=pl.BlockSpec((1, 1), lambda r: (0, 0)),
        out_shape=jax.ShapeDtypeStruct((1, 1), jnp.float32),
        compiler_params=pltpu.CompilerParams(
            dimension_semantics=("arbitrary",)),
    )(x_hat, x)


def kernel(x, W_enc, b_enc, W_dec, b_dec):
    pre, cmax = _encode(x, W_enc, b_enc, b_dec)
    z, _vals, _idx = _sc_topk(pre, cmax)
    x_hat = _decode(z, W_dec, b_dec)
    loss = _loss(x_hat, x)
    return (loss.reshape(()), x_hat, z)


# speedup vs baseline: 7.1611x; 1.2362x over previous
import functools

import jax
import jax.numpy as jnp
from jax import lax
from jax.experimental import pallas as pl
from jax.experimental.pallas import tpu as pltpu
from jax.experimental.pallas import tpu_sc as plsc

D_IN = 2048
D_SAE = 32768
KTOP = 64
B = 2048


BL = 1024
NLB = D_SAE // BL
NCHUNK = D_SAE // 8


def _encode_body(x_ref, w_ref, be_ref, bd_ref, pre_ref, cm_ref):
    xc = x_ref[...] - bd_ref[...]
    acc = lax.dot_general(xc, w_ref[...], (((1,), (1,)), ((), ())),
                          preferred_element_type=jnp.float32)
    pre = acc + be_ref[...]
    pre_ref[...] = pre
    cm = pre[:, 0:128]
    for m in range(1, 8):
        cm = jnp.maximum(cm, pre[:, 128 * m:128 * (m + 1)])
    cm_ref[...] = cm


def _encode(x, w_enc, b_enc, b_dec):
    return pl.pallas_call(
        _encode_body,
        grid=(2, NLB),
        in_specs=[
            pl.BlockSpec((B // 2, D_IN), lambda r, l: (r, 0)),
            pl.BlockSpec((BL, D_IN), lambda r, l: (l, 0)),
            pl.BlockSpec((1, BL), lambda r, l: (0, l)),
            pl.BlockSpec((1, D_IN), lambda r, l: (0, 0)),
        ],
        out_specs=[
            pl.BlockSpec((B // 2, BL), lambda r, l: (r, l)),
            pl.BlockSpec((B // 2, 128), lambda r, l: (r, l)),
        ],
        out_shape=[
            jax.ShapeDtypeStruct((B, D_SAE), jnp.float32),
            jax.ShapeDtypeStruct((B, NCHUNK), jnp.float32),
        ],
        compiler_params=pltpu.CompilerParams(
            dimension_semantics=("arbitrary", "arbitrary")),
    )(x, w_enc, b_enc.reshape(1, D_SAE), b_dec.reshape(1, D_IN))


NC = 2
NS = 16
NW = NC * NS
RPW = B // NW
CAP = 128
NEL = CAP * 8

_I32 = jnp.int32


def _mono(v):
    b = lax.bitcast_convert_type(v, _I32)
    return b ^ (jnp.right_shift(b, 31) & _I32(0x7FFFFFFF))


def _vext(vec, lane):
    return jnp.sum(jnp.where(lax.iota(_I32, 16) == lane, vec,
                             jnp.zeros((16,), vec.dtype)))


def _zero_hist(hist):
    z16 = jnp.zeros((16,), _I32)
    for i in range(16):
        hist[pl.ds(i * 16, 16)] = z16


def _scan_hist(hist, target):
    def ph1(i, c):
        acc, cross, accb = c
        t = jnp.sum(hist[pl.ds(i * 16, 16)])
        hit = jnp.logical_and(acc + t > target, cross < 0)
        cross = jnp.where(hit, i, cross)
        accb = jnp.where(hit, acc, accb)
        return acc + t, cross, accb

    _, cross, accb = lax.fori_loop(0, 16, ph1, (_I32(0), _I32(-1), _I32(0)),
                                   unroll=4)
    h = hist[pl.ds(cross * 16, 16)]
    cs = plsc.cumsum(h)
    m = (accb + cs) > target
    lane = jnp.max(plsc.all_reduce_ffs(m))
    b = cross * 16 + lane
    cb = accb + _vext(cs, lane)
    hb = _vext(h, lane)
    return b, cb, hb


def _histogram_pass(hist, n_vregs, get_bucket_mask):
    def body(t, carry):
        bkt, msk = get_bucket_mask(t)
        cnt, last = plsc.scan_count(bkt, mask=msk)
        plsc.addupdate_scatter(hist, [bkt], cnt, mask=last)
        return carry

    lax.fori_loop(0, n_vregs, body, 0, unroll=8)


def _sc_topk_kernel(pre_hbm, cmax_hbm, z_hbm, vals_hbm, idx_hbm,
                    cm_v, s_chunk, hist, cand_ch, eidx, colidx, s_elem,
                    cand_v, vals_buf, idx_buf, z_buf,
                    sem_g0, sem_g1, sem_z0, sem_z1):
    cid = lax.axis_index("c")
    sid = lax.axis_index("s")
    wid = sid * NC + cid
    row0 = wid * RPW
    iota = lax.iota(_I32, 16)
    zero16f = jnp.zeros((16,), jnp.float32)

    def zb(i, c):
        z_buf[pl.ds(i * 16, 16)] = zero16f
        return c
    lax.fori_loop(0, 2 * D_SAE // 16, zb, 0)

    def chunk_stage(r, gbuf):
        pltpu.sync_copy(cmax_hbm.at[r], cm_v)

        _zero_hist(hist)

        def l1(t):
            s = _mono(cm_v[pl.ds(t * 16, 16)])
            s_chunk[pl.ds(t * 16, 16)] = s
            b = jnp.right_shift(s, 24) + 128
            return b, None
        _histogram_pass(hist, NCHUNK // 16, l1)
        b1, c1, h1 = _scan_hist(hist, _I32(NCHUNK - KTOP))
        a1 = NCHUNK - c1

        _zero_hist(hist)

        def l2(t):
            s = s_chunk[pl.ds(t * 16, 16)]
            msk = (jnp.right_shift(s, 24) + 128) == b1
            b = jnp.right_shift(s, 16) & 0xFF
            return b, msk
        _histogram_pass(hist, NCHUNK // 16, l2)
        b2, _, _ = _scan_hist(hist, h1 - (KTOP - a1))
        floor16 = jnp.left_shift(b1 - 128, 24) | jnp.left_shift(b2, 16)

        for j in range(CAP // 16):
            cand_ch[pl.ds(j * 16, 16)] = iota + j * 16

        def sel(t, n):
            s = s_chunk[pl.ds(t * 16, 16)]
            m = s >= floor16
            plsc.store_compressed(cand_ch.at[pl.ds(n, 16)],
                                  iota + t * 16, mask=m)
            return n + jnp.max(plsc.all_reduce_population_count(m))
        n_ch = lax.fori_loop(0, NCHUNK // 16, sel, _I32(0), unroll=4)
        n_ch = jnp.minimum(n_ch, _I32(CAP))

        rbase = r * D_SAE
        for j in range(CAP // 16):
            c = cand_ch[pl.ds(j * 16, 16)]
            base = jnp.left_shift(jnp.right_shift(c, 7), 10) + (c & 127)
            for m in range(8):
                t = j * 8 + m
                col = base + 128 * m
                colidx[pl.ds(gbuf * NEL + t * 16, 16)] = col
                eidx[pl.ds(gbuf * NEL + t * 16, 16)] = col + rbase
        return n_ch

    def elem_stage(r, n_ch, gbuf, p, sem_z):
        n_el = n_ch * 8
        eoff = gbuf * NEL

        def valid_of(t):
            j = t // 8 if isinstance(t, int) else jnp.right_shift(t, 3)
            return (j * 16 + iota) < n_ch

        _zero_hist(hist)

        def l1(t):
            v = cand_v[pl.ds(eoff + t * 16, 16)]
            s = _mono(v)
            s_elem[pl.ds(eoff + t * 16, 16)] = s
            return jnp.right_shift(s, 24) + 128, valid_of(t)
        _histogram_pass(hist, NEL // 16, l1)
        b1, c1, h1 = _scan_hist(hist, n_el - KTOP)
        a1 = n_el - c1

        _zero_hist(hist)

        def l2(t):
            s = s_elem[pl.ds(eoff + t * 16, 16)]
            msk = jnp.logical_and(valid_of(t),
                                  (jnp.right_shift(s, 24) + 128) == b1)
            return jnp.right_shift(s, 16) & 0xFF, msk
        _histogram_pass(hist, NEL // 16, l2)
        b2, c2, h2 = _scan_hist(hist, h1 - (KTOP - a1))
        a2 = h1 - c2
        pre16 = jnp.left_shift(b1 - 128, 8) | b2

        _zero_hist(hist)

        def l3(t):
            s = s_elem[pl.ds(eoff + t * 16, 16)]
            msk = jnp.logical_and(valid_of(t),
                                  jnp.right_shift(s, 16) == pre16)
            return jnp.right_shift(s, 8) & 0xFF, msk
        _histogram_pass(hist, NEL // 16, l3)
        b3, c3, h3 = _scan_hist(hist, h2 - (KTOP - a1 - a2))
        a3 = h2 - c3
        pre24 = jnp.left_shift(pre16, 8) | b3

        _zero_hist(hist)

        def l4(t):
            s = s_elem[pl.ds(eoff + t * 16, 16)]
            msk = jnp.logical_and(valid_of(t),
                                  jnp.right_shift(s, 8) == pre24)
            return s & 0xFF, msk
        _histogram_pass(hist, NEL // 16, l4)
        b4, _, _ = _scan_hist(hist, h3 - (KTOP - a1 - a2 - a3))
        s_star = jnp.left_shift(pre24, 8) | b4

        @pl.when(r - row0 >= 2)
        def _():
            pltpu.make_async_copy(z_buf.at[pl.ds(p * D_SAE, D_SAE)],
                                  z_hbm.at[r - 2], sem_z).wait()
            for j in range(KTOP // 16):
                ii = idx_buf[pl.ds(p * 80 + j * 16, 16)]
                plsc.store_scatter(z_buf, [ii + p * D_SAE], zero16f)

        def sel(t, n):
            s = s_elem[pl.ds(eoff + t * 16, 16)]
            m = jnp.logical_and(valid_of(t), s >= s_star)
            pc = plsc.cumsum(m.astype(_I32))
            keep = jnp.logical_and(m, (n + pc) <= KTOP)
            v = cand_v[pl.ds(eoff + t * 16, 16)]
            plsc.store_compressed(vals_buf.at[pl.ds(n, 16)],
                                  jnp.maximum(v, 0.0), mask=keep)
            plsc.store_compressed(idx_buf.at[pl.ds(p * 80 + n, 16)],
                                  colidx[pl.ds(eoff + t * 16, 16)],
                                  mask=keep)
            return n + jnp.sum(keep.astype(_I32))
        lax.fori_loop(0, NEL // 16, sel, _I32(0))

        for j in range(KTOP // 16):
            ii = idx_buf[pl.ds(p * 80 + j * 16, 16)]
            vv = vals_buf[pl.ds(j * 16, 16)]
            plsc.store_scatter(z_buf, [ii + p * D_SAE], vv)

        pltpu.async_copy(z_buf.at[pl.ds(p * D_SAE, D_SAE)],
                         z_hbm.at[r], sem_z)
        pltpu.sync_copy(vals_buf.at[pl.ds(0, KTOP)],
                        vals_hbm.at[pl.ds(r * KTOP, KTOP)])
        pltpu.sync_copy(idx_buf.at[pl.ds(p * 80, KTOP)],
                        idx_hbm.at[pl.ds(r * KTOP, KTOP)])

    def pair(rr, c):
        r0 = row0 + 2 * rr
        r1 = r0 + 1
        n0 = chunk_stage(r0, 0)
        cpy0 = pltpu.async_copy(pre_hbm.at[eidx.at[pl.ds(0, NEL)]],
                                cand_v.at[pl.ds(0, NEL)], sem_g0)
        n1 = chunk_stage(r1, 1)
        cpy1 = pltpu.async_copy(pre_hbm.at[eidx.at[pl.ds(NEL, NEL)]],
                                cand_v.at[pl.ds(NEL, NEL)], sem_g1)
        cpy0.wait()
        elem_stage(r0, n0, 0, 0, sem_z0)
        cpy1.wait()
        elem_stage(r1, n1, 1, 1, sem_z1)
        return c

    lax.fori_loop(0, RPW // 2, pair, 0)

    pltpu.make_async_copy(z_buf.at[pl.ds(0, D_SAE)],
                          z_hbm.at[row0 + RPW - 2], sem_z0).wait()
    pltpu.make_async_copy(z_buf.at[pl.ds(D_SAE, D_SAE)],
                          z_hbm.at[row0 + RPW - 1], sem_z1).wait()


def _sc_topk(pre, cmax):
    mesh = plsc.VectorSubcoreMesh(core_axis_name="c", subcore_axis_name="s",
                                  num_cores=NC, num_subcores=NS)
    kfn = pl.kernel(
        _sc_topk_kernel,
        out_type=[
            jax.ShapeDtypeStruct((B, D_SAE), jnp.float32),
            jax.ShapeDtypeStruct((B * KTOP,), jnp.float32),
            jax.ShapeDtypeStruct((B * KTOP,), jnp.int32),
        ],
        mesh=mesh,
        compiler_params=pltpu.CompilerParams(needs_layout_passes=False),
        scratch_types=[
            pltpu.VMEM((NCHUNK,), jnp.float32),
            pltpu.VMEM((NCHUNK,), jnp.int32),
            pltpu.VMEM((256,), jnp.int32),
            pltpu.VMEM((NCHUNK + 16,), jnp.int32),
            pltpu.VMEM((2 * NEL,), jnp.int32),
            pltpu.VMEM((2 * NEL,), jnp.int32),
            pltpu.VMEM((2 * NEL,), jnp.int32),
            pltpu.VMEM((2 * NEL,), jnp.float32),
            pltpu.VMEM((80,), jnp.float32),
            pltpu.VMEM((160,), jnp.int32),
            pltpu.VMEM((2 * D_SAE,), jnp.float32),
            pltpu.SemaphoreType.DMA,
            pltpu.SemaphoreType.DMA,
            pltpu.SemaphoreType.DMA,
            pltpu.SemaphoreType.DMA,
        ],
    )
    zf, vf, inf = kfn(pre.reshape(-1), cmax)
    return (zf, vf.reshape(B, KTOP), inf.reshape(B, KTOP))


KB = 1024
NKB = D_SAE // KB


def _decode_body(z_ref, w_ref, bd_ref, out_ref):
    kb = pl.program_id(0)

    @pl.when(kb == 0)
    def _():
        out_ref[...] = jnp.broadcast_to(bd_ref[...], (B, D_IN))

    out_ref[...] += lax.dot_general(
        z_ref[...].astype(jnp.bfloat16), w_ref[...].astype(jnp.bfloat16),
        (((1,), (1,)), ((), ())), preferred_element_type=jnp.float32)


def _decode(z, w_dec, b_dec):
    return pl.pallas_call(
        _decode_body,
        grid=(NKB,),
        in_specs=[
            pl.BlockSpec((B, KB), lambda k: (0, k)),
            pl.BlockSpec((D_IN, KB), lambda k: (0, k)),
            pl.BlockSpec((1, D_IN), lambda k: (0, 0)),
        ],
        out_specs=pl.BlockSpec((B, D_IN), lambda k: (0, 0)),
        out_shape=jax.ShapeDtypeStruct((B, D_IN), jnp.float32),
        compiler_params=pltpu.CompilerParams(
            dimension_semantics=("arbitrary",)),
    )(z, w_dec, b_dec.reshape(1, D_IN))


LRB = 256
NLRB = B // LRB


def _loss_body(xh_ref, x_ref, out_ref):
    rb = pl.program_id(0)
    d = xh_ref[...] - x_ref[...]
    s = jnp.sum(d * d).reshape(1, 1)

    @pl.when(rb == 0)
    def _():
        out_ref[...] = jnp.zeros((1, 1), jnp.float32)

    out_ref[...] += s

    @pl.when(rb == NLRB - 1)
    def _():
        out_ref[...] = out_ref[...] / B


def _loss(x_hat, x):
    return pl.pallas_call(
        _loss_body,
        grid=(NLRB,),
        in_specs=[
            pl.BlockSpec((LRB, D_IN), lambda r: (r, 0)),
            pl.BlockSpec((LRB, D_IN), lambda r: (r, 0)),
        ],
        out_specs=pl.BlockSpec((1, 1), lambda r: (0, 0)),
        out_shape=jax.ShapeDtypeStruct((1, 1), jnp.float32),
        compiler_params=pltpu.CompilerParams(
            dimension_semantics=("arbitrary",)),
    )(x_hat, x)


def kernel(x, W_enc, b_enc, W_dec, b_dec):
    pre, cmax = _encode(x, W_enc, b_enc, b_dec)
    z, _vals, _idx = _sc_topk(pre, cmax)
    x_hat = _decode(z, W_dec, b_dec)
    loss = _loss(x_hat, x)
    return (loss.reshape(()), x_hat, z)


# speedup vs baseline: 8.2122x; 1.1468x over previous
import functools

import jax
import jax.numpy as jnp
from jax import lax
from jax.experimental import pallas as pl
from jax.experimental.pallas import tpu as pltpu
from jax.experimental.pallas import tpu_sc as plsc

D_IN = 2048
D_SAE = 32768
KTOP = 64
B = 2048


BL = 2048
NLB = D_SAE // BL
NCHUNK = D_SAE // 16


def _encode_body(x_ref, w_ref, be_ref, bd_ref, pre_ref, cm_ref):
    xc = x_ref[...] - bd_ref[...]
    acc = lax.dot_general(xc, w_ref[...], (((1,), (1,)), ((), ())),
                          preferred_element_type=jnp.float32)
    pre = acc + be_ref[...]
    pre_ref[...] = pre
    cm = pre[:, 0:128]
    for m in range(1, 16):
        cm = jnp.maximum(cm, pre[:, 128 * m:128 * (m + 1)])
    cm_ref[...] = cm


def _encode(x, w_enc, b_enc, b_dec):
    return pl.pallas_call(
        _encode_body,
        grid=(4, NLB),
        in_specs=[
            pl.BlockSpec((B // 4, D_IN), lambda r, l: (r, 0)),
            pl.BlockSpec((BL, D_IN), lambda r, l: (l, 0)),
            pl.BlockSpec((1, BL), lambda r, l: (0, l)),
            pl.BlockSpec((1, D_IN), lambda r, l: (0, 0)),
        ],
        out_specs=[
            pl.BlockSpec((B // 4, BL), lambda r, l: (r, l)),
            pl.BlockSpec((B // 4, 128), lambda r, l: (r, l)),
        ],
        out_shape=[
            jax.ShapeDtypeStruct((B, D_SAE), jnp.float32),
            jax.ShapeDtypeStruct((B, NCHUNK), jnp.float32),
        ],
        compiler_params=pltpu.CompilerParams(
            dimension_semantics=("arbitrary", "arbitrary")),
    )(x, w_enc, b_enc.reshape(1, D_SAE), b_dec.reshape(1, D_IN))


NC = 2
NS = 16
NW = NC * NS
RPW = B // NW
CAP = 96
NEL = CAP * 16

_I32 = jnp.int32


def _mono(v):
    b = lax.bitcast_convert_type(v, _I32)
    return b ^ (jnp.right_shift(b, 31) & _I32(0x7FFFFFFF))


def _vext(vec, lane):
    return jnp.sum(jnp.where(lax.iota(_I32, 16) == lane, vec,
                             jnp.zeros((16,), vec.dtype)))


def _zero_hist(hist):
    z16 = jnp.zeros((16,), _I32)
    for i in range(256):
        hist[pl.ds(i * 16, 16)] = z16


_LANE_OFF = None


def _scan_hist(hist, target):
    def group_tot(g):
        v = hist[pl.ds(g * 16, 16)]
        for l in range(1, 16):
            v = v + hist[pl.ds(l * 256 + g * 16, 16)]
        return v

    def ph1(i, c):
        acc, cross, accb = c
        t = jnp.sum(group_tot(i))
        hit = jnp.logical_and(acc + t > target, cross < 0)
        cross = jnp.where(hit, i, cross)
        accb = jnp.where(hit, acc, accb)
        return acc + t, cross, accb

    _, cross, accb = lax.fori_loop(0, 16, ph1, (_I32(0), _I32(-1), _I32(0)),
                                   unroll=2)
    h = group_tot(cross)
    cs = plsc.cumsum(h)
    m = (accb + cs) > target
    lane = jnp.max(plsc.all_reduce_ffs(m))
    b = cross * 16 + lane
    cb = accb + _vext(cs, lane)
    hb = _vext(h, lane)
    return b, cb, hb


def _histogram_pass(hist, n_vregs, get_bucket_mask):
    lane_off = lax.iota(_I32, 16) * 256
    ones16 = jnp.ones((16,), _I32)

    def body(t, carry):
        bkt, msk = get_bucket_mask(t)
        plsc.addupdate_scatter(hist, [bkt + lane_off], ones16, mask=msk)
        return carry

    if isinstance(n_vregs, int):
        lax.fori_loop(0, n_vregs, body, 0, unroll=8)
    else:
        lax.fori_loop(0, n_vregs, body, 0)


def _sc_topk_kernel(pre_hbm, cmax_hbm, z_hbm, vals_hbm, idx_hbm,
                    cm_v, s_chunk, hist, cand_ch, eidx, colidx, s_elem,
                    cand_v, vals_buf, idx_buf, z_buf,
                    sem_g0, sem_g1, sem_z0, sem_z1):
    cid = lax.axis_index("c")
    sid = lax.axis_index("s")
    wid = sid * NC + cid
    row0 = wid * RPW
    iota = lax.iota(_I32, 16)
    zero16f = jnp.zeros((16,), jnp.float32)

    def zb(i, c):
        z_buf[pl.ds(i * 16, 16)] = zero16f
        return c
    lax.fori_loop(0, 2 * D_SAE // 16, zb, 0)

    def chunk_stage(r, gbuf):
        pltpu.sync_copy(cmax_hbm.at[r], cm_v)

        _zero_hist(hist)

        def l1(t):
            s = _mono(cm_v[pl.ds(t * 16, 16)])
            s_chunk[pl.ds(t * 16, 16)] = s
            b = jnp.right_shift(s, 24) + 128
            return b, None
        _histogram_pass(hist, NCHUNK // 16, l1)
        b1, c1, h1 = _scan_hist(hist, _I32(NCHUNK - KTOP))
        a1 = NCHUNK - c1

        _zero_hist(hist)

        def l2(t):
            s = s_chunk[pl.ds(t * 16, 16)]
            msk = (jnp.right_shift(s, 24) + 128) == b1
            b = jnp.right_shift(s, 16) & 0xFF
            return b, msk
        _histogram_pass(hist, NCHUNK // 16, l2)
        b2, _, _ = _scan_hist(hist, h1 - (KTOP - a1))
        floor16 = jnp.left_shift(b1 - 128, 24) | jnp.left_shift(b2, 16)

        for j in range(CAP // 16):
            cand_ch[pl.ds(j * 16, 16)] = iota + j * 16

        def sel(t, n):
            s = s_chunk[pl.ds(t * 16, 16)]
            m = s >= floor16
            plsc.store_compressed(cand_ch.at[pl.ds(n, 16)],
                                  iota + t * 16, mask=m)
            return n + jnp.max(plsc.all_reduce_population_count(m))
        n_ch = lax.fori_loop(0, NCHUNK // 16, sel, _I32(0), unroll=4)
        n_ch = jnp.minimum(n_ch, _I32(CAP))

        rbase = r * D_SAE
        for j in range(CAP // 16):
            c = cand_ch[pl.ds(j * 16, 16)]
            base = jnp.left_shift(jnp.right_shift(c, 7), 11) + (c & 127)
            for m in range(16):
                t = j * 16 + m
                col = base + 128 * m
                colidx[pl.ds(gbuf * NEL + t * 16, 16)] = col
                eidx[pl.ds(gbuf * NEL + t * 16, 16)] = col + rbase
        return n_ch

    def elem_stage(r, n_ch, gbuf, p, sem_z):
        n_el = n_ch * 16
        eoff = gbuf * NEL
        n_ev = jnp.left_shift(jnp.right_shift(n_ch + 15, 4), 4)

        def valid_of(t):
            j = jnp.right_shift(t, 4)
            return (j * 16 + iota) < n_ch

        _zero_hist(hist)

        def l1(t):
            v = cand_v[pl.ds(eoff + t * 16, 16)]
            s = _mono(v)
            s_elem[pl.ds(eoff + t * 16, 16)] = s
            return jnp.right_shift(s, 24) + 128, valid_of(t)
        _histogram_pass(hist, n_ev, l1)
        b1, c1, h1 = _scan_hist(hist, n_el - KTOP)
        a1 = n_el - c1

        _zero_hist(hist)

        def l2(t):
            s = s_elem[pl.ds(eoff + t * 16, 16)]
            msk = jnp.logical_and(valid_of(t),
                                  (jnp.right_shift(s, 24) + 128) == b1)
            return jnp.right_shift(s, 16) & 0xFF, msk
        _histogram_pass(hist, n_ev, l2)
        b2, c2, h2 = _scan_hist(hist, h1 - (KTOP - a1))
        a2 = h1 - c2
        pre16 = jnp.left_shift(b1 - 128, 8) | b2

        _zero_hist(hist)

        def l3(t):
            s = s_elem[pl.ds(eoff + t * 16, 16)]
            msk = jnp.logical_and(valid_of(t),
                                  jnp.right_shift(s, 16) == pre16)
            return jnp.right_shift(s, 8) & 0xFF, msk
        _histogram_pass(hist, n_ev, l3)
        b3, c3, h3 = _scan_hist(hist, h2 - (KTOP - a1 - a2))
        a3 = h2 - c3
        pre24 = jnp.left_shift(pre16, 8) | b3

        _zero_hist(hist)

        def l4(t):
            s = s_elem[pl.ds(eoff + t * 16, 16)]
            msk = jnp.logical_and(valid_of(t),
                                  jnp.right_shift(s, 8) == pre24)
            return s & 0xFF, msk
        _histogram_pass(hist, n_ev, l4)
        b4, _, _ = _scan_hist(hist, h3 - (KTOP - a1 - a2 - a3))
        s_star = jnp.left_shift(pre24, 8) | b4

        @pl.when(r - row0 >= 2)
        def _():
            pltpu.make_async_copy(z_buf.at[pl.ds(p * D_SAE, D_SAE)],
                                  z_hbm.at[r - 2], sem_z).wait()
            for j in range(KTOP // 16):
                ii = idx_buf[pl.ds(p * 80 + j * 16, 16)]
                plsc.store_scatter(z_buf, [ii + p * D_SAE], zero16f)

        def sel(t, n):
            s = s_elem[pl.ds(eoff + t * 16, 16)]
            m = jnp.logical_and(valid_of(t), s >= s_star)
            pc = plsc.cumsum(m.astype(_I32))
            keep = jnp.logical_and(m, (n + pc) <= KTOP)
            v = cand_v[pl.ds(eoff + t * 16, 16)]
            plsc.store_compressed(vals_buf.at[pl.ds(n, 16)],
                                  jnp.maximum(v, 0.0), mask=keep)
            plsc.store_compressed(idx_buf.at[pl.ds(p * 80 + n, 16)],
                                  colidx[pl.ds(eoff + t * 16, 16)],
                                  mask=keep)
            return n + jnp.sum(keep.astype(_I32))
        lax.fori_loop(0, n_ev, sel, _I32(0))

        for j in range(KTOP // 16):
            ii = idx_buf[pl.ds(p * 80 + j * 16, 16)]
            vv = vals_buf[pl.ds(j * 16, 16)]
            plsc.store_scatter(z_buf, [ii + p * D_SAE], vv)

        pltpu.async_copy(z_buf.at[pl.ds(p * D_SAE, D_SAE)],
                         z_hbm.at[r], sem_z)
        pltpu.sync_copy(vals_buf.at[pl.ds(0, KTOP)],
                        vals_hbm.at[pl.ds(r * KTOP, KTOP)])
        pltpu.sync_copy(idx_buf.at[pl.ds(p * 80, KTOP)],
                        idx_hbm.at[pl.ds(r * KTOP, KTOP)])

    def pair(rr, c):
        r0 = row0 + 2 * rr
        r1 = r0 + 1
        n0 = chunk_stage(r0, 0)
        cpy0 = pltpu.async_copy(pre_hbm.at[eidx.at[pl.ds(0, NEL)]],
                                cand_v.at[pl.ds(0, NEL)], sem_g0)
        n1 = chunk_stage(r1, 1)
        cpy1 = pltpu.async_copy(pre_hbm.at[eidx.at[pl.ds(NEL, NEL)]],
                                cand_v.at[pl.ds(NEL, NEL)], sem_g1)
        cpy0.wait()
        elem_stage(r0, n0, 0, 0, sem_z0)
        cpy1.wait()
        elem_stage(r1, n1, 1, 1, sem_z1)
        return c

    lax.fori_loop(0, RPW // 2, pair, 0)

    pltpu.make_async_copy(z_buf.at[pl.ds(0, D_SAE)],
                          z_hbm.at[row0 + RPW - 2], sem_z0).wait()
    pltpu.make_async_copy(z_buf.at[pl.ds(D_SAE, D_SAE)],
                          z_hbm.at[row0 + RPW - 1], sem_z1).wait()


def _sc_topk(pre, cmax):
    mesh = plsc.VectorSubcoreMesh(core_axis_name="c", subcore_axis_name="s",
                                  num_cores=NC, num_subcores=NS)
    kfn = pl.kernel(
        _sc_topk_kernel,
        out_type=[
            jax.ShapeDtypeStruct((B, D_SAE), jnp.float32),
            jax.ShapeDtypeStruct((B * KTOP,), jnp.float32),
            jax.ShapeDtypeStruct((B * KTOP,), jnp.int32),
        ],
        mesh=mesh,
        compiler_params=pltpu.CompilerParams(needs_layout_passes=False),
        scratch_types=[
            pltpu.VMEM((NCHUNK,), jnp.float32),
            pltpu.VMEM((NCHUNK,), jnp.int32),
            pltpu.VMEM((4096,), jnp.int32),
            pltpu.VMEM((NCHUNK + 16,), jnp.int32),
            pltpu.VMEM((2 * NEL,), jnp.int32),
            pltpu.VMEM((2 * NEL,), jnp.int32),
            pltpu.VMEM((2 * NEL,), jnp.int32),
            pltpu.VMEM((2 * NEL,), jnp.float32),
            pltpu.VMEM((80,), jnp.float32),
            pltpu.VMEM((160,), jnp.int32),
            pltpu.VMEM((2 * D_SAE,), jnp.float32),
            pltpu.SemaphoreType.DMA,
            pltpu.SemaphoreType.DMA,
            pltpu.SemaphoreType.DMA,
            pltpu.SemaphoreType.DMA,
        ],
    )
    zf, vf, inf = kfn(pre.reshape(-1), cmax)
    return (zf, vf.reshape(B, KTOP), inf.reshape(B, KTOP))


KB = 1024
NKB = D_SAE // KB


def _decode_body(z_ref, w_ref, bd_ref, out_ref):
    kb = pl.program_id(0)

    @pl.when(kb == 0)
    def _():
        out_ref[...] = jnp.broadcast_to(bd_ref[...], (B, D_IN))

    out_ref[...] += lax.dot_general(
        z_ref[...].astype(jnp.bfloat16), w_ref[...].astype(jnp.bfloat16),
        (((1,), (1,)), ((), ())), preferred_element_type=jnp.float32)


def _decode(z, w_dec, b_dec):
    return pl.pallas_call(
        _decode_body,
        grid=(NKB,),
        in_specs=[
            pl.BlockSpec((B, KB), lambda k: (0, k)),
            pl.BlockSpec((D_IN, KB), lambda k: (0, k)),
            pl.BlockSpec((1, D_IN), lambda k: (0, 0)),
        ],
        out_specs=pl.BlockSpec((B, D_IN), lambda k: (0, 0)),
        out_shape=jax.ShapeDtypeStruct((B, D_IN), jnp.float32),
        compiler_params=pltpu.CompilerParams(
            dimension_semantics=("arbitrary",)),
    )(z, w_dec, b_dec.reshape(1, D_IN))


LRB = 256
NLRB = B // LRB


def _loss_body(xh_ref, x_ref, out_ref):
    rb = pl.program_id(0)
    d = xh_ref[...] - x_ref[...]
    s = jnp.sum(d * d).reshape(1, 1)

    @pl.when(rb == 0)
    def _():
        out_ref[...] = jnp.zeros((1, 1), jnp.float32)

    out_ref[...] += s

    @pl.when(rb == NLRB - 1)
    def _():
        out_ref[...] = out_ref[...] / B


def _loss(x_hat, x):
    return pl.pallas_call(
        _loss_body,
        grid=(NLRB,),
        in_specs=[
            pl.BlockSpec((LRB, D_IN), lambda r: (r, 0)),
            pl.BlockSpec((LRB, D_IN), lambda r: (r, 0)),
        ],
        out_specs=pl.BlockSpec((1, 1), lambda r: (0, 0)),
        out_shape=jax.ShapeDtypeStruct((1, 1), jnp.float32),
        compiler_params=pltpu.CompilerParams(
            dimension_semantics=("arbitrary",)),
    )(x_hat, x)


def kernel(x, W_enc, b_enc, W_dec, b_dec):
    pre, cmax = _encode(x, W_enc, b_enc, b_dec)
    z, _vals, _idx = _sc_topk(pre, cmax)
    x_hat = _decode(z, W_dec, b_dec)
    loss = _loss(x_hat, x)
    return (loss.reshape(()), x_hat, z)


# speedup vs baseline: 9.8123x; 1.1948x over previous
import functools

import jax
import jax.numpy as jnp
from jax import lax
from jax.experimental import pallas as pl
from jax.experimental.pallas import tpu as pltpu
from jax.experimental.pallas import tpu_sc as plsc

D_IN = 2048
D_SAE = 32768
KTOP = 64
B = 2048


BL = 2048
NLB = D_SAE // BL
NCHUNK = D_SAE // 16


def _encode_body(x_ref, w_ref, be_ref, bd_ref, pre_ref, cm_ref):
    xc = x_ref[...] - bd_ref[...]
    acc = lax.dot_general(xc, w_ref[...], (((1,), (1,)), ((), ())),
                          preferred_element_type=jnp.float32)
    pre = acc + be_ref[...]
    pre_ref[...] = pre
    cm = pre[:, 0:128]
    for m in range(1, 16):
        cm = jnp.maximum(cm, pre[:, 128 * m:128 * (m + 1)])
    cm_ref[...] = cm


def _encode(x, w_enc, b_enc, b_dec):
    return pl.pallas_call(
        _encode_body,
        grid=(4, NLB),
        in_specs=[
            pl.BlockSpec((B // 4, D_IN), lambda r, l: (r, 0)),
            pl.BlockSpec((BL, D_IN), lambda r, l: (l, 0)),
            pl.BlockSpec((1, BL), lambda r, l: (0, l)),
            pl.BlockSpec((1, D_IN), lambda r, l: (0, 0)),
        ],
        out_specs=[
            pl.BlockSpec((B // 4, BL), lambda r, l: (r, l)),
            pl.BlockSpec((B // 4, 128), lambda r, l: (r, l)),
        ],
        out_shape=[
            jax.ShapeDtypeStruct((B, D_SAE), jnp.float32),
            jax.ShapeDtypeStruct((B, NCHUNK), jnp.float32),
        ],
        compiler_params=pltpu.CompilerParams(
            dimension_semantics=("arbitrary", "arbitrary")),
    )(x, w_enc, b_enc.reshape(1, D_SAE), b_dec.reshape(1, D_IN))


NC = 2
NS = 16
NW = NC * NS
RPW = B // NW
CAP = 96
NEL = CAP * 16

_I32 = jnp.int32


def _mono(v):
    b = lax.bitcast_convert_type(v, _I32)
    return b ^ (jnp.right_shift(b, 31) & _I32(0x7FFFFFFF))


def _vext(vec, lane):
    return jnp.sum(jnp.where(lax.iota(_I32, 16) == lane, vec,
                             jnp.zeros((16,), vec.dtype)))


def _zero_hist(hist):
    z16 = jnp.zeros((16,), _I32)
    for i in range(256):
        hist[pl.ds(i * 16, 16)] = z16


_LANE_OFF = None


def _scan_hist(hist, target):
    def group_tot(g):
        v = hist[pl.ds(g * 16, 16)]
        for l in range(1, 16):
            v = v + hist[pl.ds(l * 256 + g * 16, 16)]
        return v

    def ph1(i, c):
        acc, cross, accb = c
        t = jnp.sum(group_tot(i))
        hit = jnp.logical_and(acc + t > target, cross < 0)
        cross = jnp.where(hit, i, cross)
        accb = jnp.where(hit, acc, accb)
        return acc + t, cross, accb

    _, cross, accb = lax.fori_loop(0, 16, ph1, (_I32(0), _I32(-1), _I32(0)),
                                   unroll=2)
    h = group_tot(cross)
    cs = plsc.cumsum(h)
    m = (accb + cs) > target
    lane = jnp.max(plsc.all_reduce_ffs(m))
    b = cross * 16 + lane
    cb = accb + _vext(cs, lane)
    hb = _vext(h, lane)
    return b, cb, hb


def _histogram_pass(hist, n_vregs, get_bucket_mask):
    lane_off = lax.iota(_I32, 16) * 256
    ones16 = jnp.ones((16,), _I32)

    def body(t, carry):
        bkt, msk = get_bucket_mask(t)
        plsc.addupdate_scatter(hist, [bkt + lane_off], ones16, mask=msk)
        return carry

    if isinstance(n_vregs, int):
        lax.fori_loop(0, n_vregs, body, 0, unroll=8)
    else:
        lax.fori_loop(0, n_vregs, body, 0)


def _sc_topk_kernel(pre_hbm, cmax_hbm, z_hbm, vals_hbm, idx_hbm,
                    cm_v, s_chunk, hist, cand_ch, eidx, colidx, s_f,
                    vals_f, cols_f, cand_v, vals_buf, idx_buf, z_buf,
                    sem_g0, sem_g1, sem_z0, sem_z1):
    cid = lax.axis_index("c")
    sid = lax.axis_index("s")
    wid = sid * NC + cid
    row0 = wid * RPW
    iota = lax.iota(_I32, 16)
    zero16f = jnp.zeros((16,), jnp.float32)

    def zb(i, c):
        z_buf[pl.ds(i * 16, 16)] = zero16f
        return c
    lax.fori_loop(0, 2 * D_SAE // 16, zb, 0)

    def chunk_stage(r, gbuf):
        pltpu.sync_copy(cmax_hbm.at[r], cm_v)

        _zero_hist(hist)

        def l1(t):
            s = _mono(cm_v[pl.ds(t * 16, 16)])
            s_chunk[pl.ds(t * 16, 16)] = s
            b = jnp.right_shift(s, 24) + 128
            return b, None
        _histogram_pass(hist, NCHUNK // 16, l1)
        b1, c1, h1 = _scan_hist(hist, _I32(NCHUNK - KTOP))
        a1 = NCHUNK - c1

        _zero_hist(hist)

        def l2(t):
            s = s_chunk[pl.ds(t * 16, 16)]
            msk = (jnp.right_shift(s, 24) + 128) == b1
            b = jnp.right_shift(s, 16) & 0xFF
            return b, msk
        _histogram_pass(hist, NCHUNK // 16, l2)
        b2, _, _ = _scan_hist(hist, h1 - (KTOP - a1))
        floor16 = jnp.left_shift(b1 - 128, 24) | jnp.left_shift(b2, 16)

        for j in range(CAP // 16):
            cand_ch[pl.ds(j * 16, 16)] = iota + j * 16

        def sel(t, n):
            s = s_chunk[pl.ds(t * 16, 16)]
            m = s >= floor16
            plsc.store_compressed(cand_ch.at[pl.ds(n, 16)],
                                  iota + t * 16, mask=m)
            return n + jnp.max(plsc.all_reduce_population_count(m))
        n_ch = lax.fori_loop(0, NCHUNK // 16, sel, _I32(0), unroll=4)
        n_ch = jnp.minimum(n_ch, _I32(CAP))

        rbase = r * D_SAE
        for j in range(CAP // 16):
            c = cand_ch[pl.ds(j * 16, 16)]
            base = jnp.left_shift(jnp.right_shift(c, 7), 11) + (c & 127)
            for m in range(16):
                t = j * 16 + m
                col = base + 128 * m
                colidx[pl.ds(gbuf * NEL + t * 16, 16)] = col
                eidx[pl.ds(gbuf * NEL + t * 16, 16)] = col + rbase
        return n_ch, floor16

    def elem_stage(r, n_ch, floor16, gbuf, p, sem_z):
        eoff = gbuf * NEL
        n_ev = jnp.left_shift(jnp.right_shift(n_ch + 15, 4), 4)
        int_min = jnp.full((16,), _I32(-2147483648))

        for i in range(9):
            s_f[pl.ds(i * 16, 16)] = int_min

        def passA(t, n):
            v = cand_v[pl.ds(eoff + t * 16, 16)]
            s = _mono(v)
            j = jnp.right_shift(t, 4)
            valid = (j * 16 + iota) < n_ch
            m = jnp.logical_and(valid, s >= floor16)
            off = jnp.minimum(n, _I32(128))
            plsc.store_compressed(s_f.at[pl.ds(off, 16)], s, mask=m)
            plsc.store_compressed(vals_f.at[pl.ds(off, 16)], v, mask=m)
            plsc.store_compressed(cols_f.at[pl.ds(off, 16)],
                                  colidx[pl.ds(eoff + t * 16, 16)], mask=m)
            return n + jnp.max(plsc.all_reduce_population_count(m))
        lax.fori_loop(0, n_ev, passA, _I32(0))

        sv = [s_f[pl.ds(j * 16, 16)] for j in range(8)]
        t_u = jnp.zeros((16,), _I32)
        k64 = jnp.full((16,), _I32(KTOP))
        for bit in range(31, -1, -1):
            trial_u = t_u | _I32(1 << bit) if bit < 31 else \
                t_u | _I32(-2147483648)
            trial_s = trial_u ^ _I32(-2147483648)
            cnt = jnp.zeros((16,), _I32)
            for j in range(8):
                cnt = cnt + plsc.all_reduce_population_count(sv[j] >= trial_s)
            t_u = jnp.where(cnt >= k64, trial_u, t_u)
        s_star = t_u ^ _I32(-2147483648)

        @pl.when(r - row0 >= 2)
        def _():
            pltpu.make_async_copy(z_buf.at[pl.ds(p * D_SAE, D_SAE)],
                                  z_hbm.at[r - 2], sem_z).wait()
            for j in range(KTOP // 16):
                ii = idx_buf[pl.ds(p * 80 + j * 16, 16)]
                plsc.store_scatter(z_buf, [ii + p * D_SAE], zero16f)


        def sel2(t, n):
            s = s_f[pl.ds(t * 16, 16)]
            m = s >= s_star
            pc = plsc.cumsum(m.astype(_I32))
            keep = jnp.logical_and(m, (n + pc) <= KTOP)
            v = vals_f[pl.ds(t * 16, 16)]
            plsc.store_compressed(vals_buf.at[pl.ds(n, 16)],
                                  jnp.maximum(v, 0.0), mask=keep)
            plsc.store_compressed(idx_buf.at[pl.ds(p * 80 + n, 16)],
                                  cols_f[pl.ds(t * 16, 16)], mask=keep)
            return n + jnp.sum(keep.astype(_I32))
        lax.fori_loop(0, 8, sel2, _I32(0), unroll=2)

        for j in range(KTOP // 16):
            ii = idx_buf[pl.ds(p * 80 + j * 16, 16)]
            vv = vals_buf[pl.ds(j * 16, 16)]
            plsc.store_scatter(z_buf, [ii + p * D_SAE], vv)

        pltpu.async_copy(z_buf.at[pl.ds(p * D_SAE, D_SAE)],
                         z_hbm.at[r], sem_z)
        pltpu.sync_copy(vals_buf.at[pl.ds(0, KTOP)],
                        vals_hbm.at[pl.ds(r * KTOP, KTOP)])
        pltpu.sync_copy(idx_buf.at[pl.ds(p * 80, KTOP)],
                        idx_hbm.at[pl.ds(r * KTOP, KTOP)])

    def pair(rr, c):
        r0 = row0 + 2 * rr
        r1 = r0 + 1
        n0, f0 = chunk_stage(r0, 0)
        cpy0 = pltpu.async_copy(pre_hbm.at[eidx.at[pl.ds(0, NEL)]],
                                cand_v.at[pl.ds(0, NEL)], sem_g0)
        n1, f1 = chunk_stage(r1, 1)
        cpy1 = pltpu.async_copy(pre_hbm.at[eidx.at[pl.ds(NEL, NEL)]],
                                cand_v.at[pl.ds(NEL, NEL)], sem_g1)
        cpy0.wait()
        elem_stage(r0, n0, f0, 0, 0, sem_z0)
        cpy1.wait()
        elem_stage(r1, n1, f1, 1, 1, sem_z1)
        return c

    lax.fori_loop(0, RPW // 2, pair, 0)

    pltpu.make_async_copy(z_buf.at[pl.ds(0, D_SAE)],
                          z_hbm.at[row0 + RPW - 2], sem_z0).wait()
    pltpu.make_async_copy(z_buf.at[pl.ds(D_SAE, D_SAE)],
                          z_hbm.at[row0 + RPW - 1], sem_z1).wait()


def _sc_topk(pre, cmax):
    mesh = plsc.VectorSubcoreMesh(core_axis_name="c", subcore_axis_name="s",
                                  num_cores=NC, num_subcores=NS)
    kfn = pl.kernel(
        _sc_topk_kernel,
        out_type=[
            jax.ShapeDtypeStruct((B, D_SAE), jnp.float32),
            jax.ShapeDtypeStruct((B * KTOP,), jnp.float32),
            jax.ShapeDtypeStruct((B * KTOP,), jnp.int32),
        ],
        mesh=mesh,
        compiler_params=pltpu.CompilerParams(needs_layout_passes=False),
        scratch_types=[
            pltpu.VMEM((NCHUNK,), jnp.float32),
            pltpu.VMEM((NCHUNK,), jnp.int32),
            pltpu.VMEM((4096,), jnp.int32),
            pltpu.VMEM((NCHUNK + 16,), jnp.int32),
            pltpu.VMEM((2 * NEL,), jnp.int32),
            pltpu.VMEM((2 * NEL,), jnp.int32),
            pltpu.VMEM((144,), jnp.int32),
            pltpu.VMEM((144,), jnp.float32),
            pltpu.VMEM((144,), jnp.int32),
            pltpu.VMEM((2 * NEL,), jnp.float32),
            pltpu.VMEM((80,), jnp.float32),
            pltpu.VMEM((160,), jnp.int32),
            pltpu.VMEM((2 * D_SAE,), jnp.float32),
            pltpu.SemaphoreType.DMA,
            pltpu.SemaphoreType.DMA,
            pltpu.SemaphoreType.DMA,
            pltpu.SemaphoreType.DMA,
        ],
    )
    zf, vf, inf = kfn(pre.reshape(-1), cmax)
    return (zf, vf.reshape(B, KTOP), inf.reshape(B, KTOP))


KB = 1024
NKB = D_SAE // KB


def _decode_body(z_ref, w_ref, bd_ref, out_ref):
    kb = pl.program_id(0)

    @pl.when(kb == 0)
    def _():
        out_ref[...] = jnp.broadcast_to(bd_ref[...], (B, D_IN))

    out_ref[...] += lax.dot_general(
        z_ref[...].astype(jnp.bfloat16), w_ref[...].astype(jnp.bfloat16),
        (((1,), (1,)), ((), ())), preferred_element_type=jnp.float32)


def _decode(z, w_dec, b_dec):
    return pl.pallas_call(
        _decode_body,
        grid=(NKB,),
        in_specs=[
            pl.BlockSpec((B, KB), lambda k: (0, k)),
            pl.BlockSpec((D_IN, KB), lambda k: (0, k)),
            pl.BlockSpec((1, D_IN), lambda k: (0, 0)),
        ],
        out_specs=pl.BlockSpec((B, D_IN), lambda k: (0, 0)),
        out_shape=jax.ShapeDtypeStruct((B, D_IN), jnp.float32),
        compiler_params=pltpu.CompilerParams(
            dimension_semantics=("arbitrary",)),
    )(z, w_dec, b_dec.reshape(1, D_IN))


LRB = 256
NLRB = B // LRB


def _loss_body(xh_ref, x_ref, out_ref):
    rb = pl.program_id(0)
    d = xh_ref[...] - x_ref[...]
    s = jnp.sum(d * d).reshape(1, 1)

    @pl.when(rb == 0)
    def _():
        out_ref[...] = jnp.zeros((1, 1), jnp.float32)

    out_ref[...] += s

    @pl.when(rb == NLRB - 1)
    def _():
        out_ref[...] = out_ref[...] / B


def _loss(x_hat, x):
    return pl.pallas_call(
        _loss_body,
        grid=(NLRB,),
        in_specs=[
            pl.BlockSpec((LRB, D_IN), lambda r: (r, 0)),
            pl.BlockSpec((LRB, D_IN), lambda r: (r, 0)),
        ],
        out_specs=pl.BlockSpec((1, 1), lambda r: (0, 0)),
        out_shape=jax.ShapeDtypeStruct((1, 1), jnp.float32),
        compiler_params=pltpu.CompilerParams(
            dimension_semantics=("arbitrary",)),
    )(x_hat, x)


def kernel(x, W_enc, b_enc, W_dec, b_dec):
    pre, cmax = _encode(x, W_enc, b_enc, b_dec)
    z, _vals, _idx = _sc_topk(pre, cmax)
    x_hat = _decode(z, W_dec, b_dec)
    loss = _loss(x_hat, x)
    return (loss.reshape(()), x_hat, z)


# speedup vs baseline: 9.8261x; 1.0014x over previous
import jax
import jax.numpy as jnp
from jax import lax
from jax.experimental import pallas as pl
from jax.experimental.pallas import tpu as pltpu
from jax.experimental.pallas import tpu_sc as plsc

D_IN = 2048
D_SAE = 32768
KTOP = 64
B = 2048


BL = 2048
NLB = D_SAE // BL
NCHUNK = D_SAE // 16


def _encode_body(x_ref, w_ref, be_ref, bd_ref, pre_ref, cm_ref):
    xc = x_ref[...] - bd_ref[...]
    acc = lax.dot_general(xc, w_ref[...], (((1,), (1,)), ((), ())),
                          preferred_element_type=jnp.float32)
    pre = acc + be_ref[...]
    pre_ref[...] = pre
    cm = pre[:, 0:128]
    for m in range(1, 16):
        cm = jnp.maximum(cm, pre[:, 128 * m:128 * (m + 1)])
    cm_ref[...] = cm


def _encode(x, w_enc, b_enc, b_dec):
    return pl.pallas_call(
        _encode_body,
        grid=(4, NLB),
        in_specs=[
            pl.BlockSpec((B // 4, D_IN), lambda r, l: (r, 0)),
            pl.BlockSpec((BL, D_IN), lambda r, l: (l, 0)),
            pl.BlockSpec((1, BL), lambda r, l: (0, l)),
            pl.BlockSpec((1, D_IN), lambda r, l: (0, 0)),
        ],
        out_specs=[
            pl.BlockSpec((B // 4, BL), lambda r, l: (r, l)),
            pl.BlockSpec((B // 4, 128), lambda r, l: (r, l)),
        ],
        out_shape=[
            jax.ShapeDtypeStruct((B, D_SAE), jnp.float32),
            jax.ShapeDtypeStruct((B, NCHUNK), jnp.float32),
        ],
        compiler_params=pltpu.CompilerParams(
            dimension_semantics=("arbitrary", "arbitrary")),
    )(x, w_enc, b_enc.reshape(1, D_SAE), b_dec.reshape(1, D_IN))


NC = 2
NS = 16
NW = NC * NS
RPW = B // NW
CAP = 96
NEL = CAP * 16

_I32 = jnp.int32


def _mono(v):
    b = lax.bitcast_convert_type(v, _I32)
    return b ^ (jnp.right_shift(b, 31) & _I32(0x7FFFFFFF))


def _vext(vec, lane):
    return jnp.sum(jnp.where(lax.iota(_I32, 16) == lane, vec,
                             jnp.zeros((16,), vec.dtype)))


def _zero_hist(hist):
    z16 = jnp.zeros((16,), _I32)
    for i in range(256):
        hist[pl.ds(i * 16, 16)] = z16


def _scan_hist(hist, target):
    def group_tot(g):
        v = hist[pl.ds(g * 16, 16)]
        for l in range(1, 16):
            v = v + hist[pl.ds(l * 256 + g * 16, 16)]
        return v

    def ph1(i, c):
        acc, cross, accb = c
        t = jnp.sum(group_tot(i))
        hit = jnp.logical_and(acc + t > target, cross < 0)
        cross = jnp.where(hit, i, cross)
        accb = jnp.where(hit, acc, accb)
        return acc + t, cross, accb

    _, cross, accb = lax.fori_loop(0, 16, ph1, (_I32(0), _I32(-1), _I32(0)),
                                   unroll=2)
    h = group_tot(cross)
    cs = plsc.cumsum(h)
    m = (accb + cs) > target
    lane = jnp.max(plsc.all_reduce_ffs(m))
    b = cross * 16 + lane
    cb = accb + _vext(cs, lane)
    hb = _vext(h, lane)
    return b, cb, hb


def _histogram_pass(hist, n_vregs, get_bucket_mask):
    lane_off = lax.iota(_I32, 16) * 256
    ones16 = jnp.ones((16,), _I32)

    def body(t, carry):
        bkt, msk = get_bucket_mask(t)
        plsc.addupdate_scatter(hist, [bkt + lane_off], ones16, mask=msk)
        return carry

    if isinstance(n_vregs, int):
        lax.fori_loop(0, n_vregs, body, 0, unroll=8)
    else:
        lax.fori_loop(0, n_vregs, body, 0)


def _sc_topk_kernel(pre_hbm, cmax_hbm, z_hbm, vals_hbm, idx_hbm,
                    cm_v, s_chunk, hist, cand_ch, eidx, colidx, s_f,
                    vals_f, cols_f, cand_v, vals_buf, idx_buf, z_buf,
                    sem_g0, sem_g1, sem_z0, sem_z1):
    cid = lax.axis_index("c")
    sid = lax.axis_index("s")
    wid = sid * NC + cid
    row0 = wid * RPW
    iota = lax.iota(_I32, 16)
    zero16f = jnp.zeros((16,), jnp.float32)

    def zb(i, c):
        z_buf[pl.ds(i * 16, 16)] = zero16f
        return c
    lax.fori_loop(0, 2 * D_SAE // 16, zb, 0)

    def chunk_stage(r, gbuf):
        pltpu.sync_copy(cmax_hbm.at[r], cm_v)

        _zero_hist(hist)

        def l1(t):
            s = _mono(cm_v[pl.ds(t * 16, 16)])
            s_chunk[pl.ds(t * 16, 16)] = s
            b = jnp.right_shift(s, 24) + 128
            return b, None
        _histogram_pass(hist, NCHUNK // 16, l1)
        b1, c1, h1 = _scan_hist(hist, _I32(NCHUNK - KTOP))
        a1 = NCHUNK - c1

        _zero_hist(hist)

        def l2(t):
            s = s_chunk[pl.ds(t * 16, 16)]
            msk = (jnp.right_shift(s, 24) + 128) == b1
            b = jnp.right_shift(s, 16) & 0xFF
            return b, msk
        _histogram_pass(hist, NCHUNK // 16, l2)
        b2, _, _ = _scan_hist(hist, h1 - (KTOP - a1))
        floor16 = jnp.left_shift(b1 - 128, 24) | jnp.left_shift(b2, 16)

        for j in range(CAP // 16):
            cand_ch[pl.ds(j * 16, 16)] = iota + j * 16

        def sel(t, n):
            s = s_chunk[pl.ds(t * 16, 16)]
            m = s >= floor16
            plsc.store_compressed(cand_ch.at[pl.ds(n, 16)],
                                  iota + t * 16, mask=m)
            return n + jnp.max(plsc.all_reduce_population_count(m))
        n_ch = lax.fori_loop(0, NCHUNK // 16, sel, _I32(0), unroll=4)
        n_ch = jnp.minimum(n_ch, _I32(CAP))

        rbase = r * D_SAE
        for j in range(CAP // 16):
            c = cand_ch[pl.ds(j * 16, 16)]
            base = jnp.left_shift(jnp.right_shift(c, 7), 11) + (c & 127)
            for m in range(16):
                t = j * 16 + m
                col = base + 128 * m
                colidx[pl.ds(gbuf * NEL + t * 16, 16)] = col
                eidx[pl.ds(gbuf * NEL + t * 16, 16)] = col + rbase
        return n_ch, floor16

    def elem_stage(r, n_ch, floor16, gbuf, p, sem_z):
        eoff = gbuf * NEL
        n_ev = jnp.left_shift(jnp.right_shift(n_ch + 15, 4), 4)
        int_min = jnp.full((16,), _I32(-2147483648))

        for i in range(9):
            s_f[pl.ds(i * 16, 16)] = int_min

        def passA(t, n):
            v = cand_v[pl.ds(eoff + t * 16, 16)]
            s = _mono(v)
            j = jnp.right_shift(t, 4)
            valid = (j * 16 + iota) < n_ch
            m = jnp.logical_and(valid, s >= floor16)
            off = jnp.minimum(n, _I32(128))
            plsc.store_compressed(s_f.at[pl.ds(off, 16)], s, mask=m)
            plsc.store_compressed(vals_f.at[pl.ds(off, 16)], v, mask=m)
            plsc.store_compressed(cols_f.at[pl.ds(off, 16)],
                                  colidx[pl.ds(eoff + t * 16, 16)], mask=m)
            return n + jnp.max(plsc.all_reduce_population_count(m))
        lax.fori_loop(0, n_ev, passA, _I32(0))

        sv = [s_f[pl.ds(j * 16, 16)] for j in range(8)]
        t_u = jnp.zeros((16,), _I32)
        k64 = jnp.full((16,), _I32(KTOP))
        for bit in range(31, -1, -1):
            trial_u = t_u | _I32(1 << bit) if bit < 31 else \
                t_u | _I32(-2147483648)
            trial_s = trial_u ^ _I32(-2147483648)
            cnt = jnp.zeros((16,), _I32)
            for j in range(8):
                cnt = cnt + plsc.all_reduce_population_count(sv[j] >= trial_s)
            t_u = jnp.where(cnt >= k64, trial_u, t_u)
        s_star = t_u ^ _I32(-2147483648)

        @pl.when(r - row0 >= 2)
        def _():
            pltpu.make_async_copy(z_buf.at[pl.ds(p * D_SAE, D_SAE)],
                                  z_hbm.at[r - 2], sem_z).wait()
            for j in range(KTOP // 16):
                ii = idx_buf[pl.ds(p * 80 + j * 16, 16)]
                plsc.store_scatter(z_buf, [ii + p * D_SAE], zero16f)


        def sel2(t, n):
            s = s_f[pl.ds(t * 16, 16)]
            m = s >= s_star
            pc = plsc.cumsum(m.astype(_I32))
            keep = jnp.logical_and(m, (n + pc) <= KTOP)
            v = vals_f[pl.ds(t * 16, 16)]
            plsc.store_compressed(vals_buf.at[pl.ds(n, 16)],
                                  jnp.maximum(v, 0.0), mask=keep)
            plsc.store_compressed(idx_buf.at[pl.ds(p * 80 + n, 16)],
                                  cols_f[pl.ds(t * 16, 16)], mask=keep)
            return n + jnp.sum(keep.astype(_I32))
        lax.fori_loop(0, 8, sel2, _I32(0), unroll=2)

        for j in range(KTOP // 16):
            ii = idx_buf[pl.ds(p * 80 + j * 16, 16)]
            vv = vals_buf[pl.ds(j * 16, 16)]
            plsc.store_scatter(z_buf, [ii + p * D_SAE], vv)

        pltpu.async_copy(z_buf.at[pl.ds(p * D_SAE, D_SAE)],
                         z_hbm.at[r], sem_z)
        pltpu.sync_copy(vals_buf.at[pl.ds(0, KTOP)],
                        vals_hbm.at[pl.ds(r * KTOP, KTOP)])
        pltpu.sync_copy(idx_buf.at[pl.ds(p * 80, KTOP)],
                        idx_hbm.at[pl.ds(r * KTOP, KTOP)])

    def pair(rr, c):
        r0 = row0 + 2 * rr
        r1 = r0 + 1
        n0, f0 = chunk_stage(r0, 0)
        cpy0 = pltpu.async_copy(pre_hbm.at[eidx.at[pl.ds(0, NEL)]],
                                cand_v.at[pl.ds(0, NEL)], sem_g0)
        n1, f1 = chunk_stage(r1, 1)
        cpy1 = pltpu.async_copy(pre_hbm.at[eidx.at[pl.ds(NEL, NEL)]],
                                cand_v.at[pl.ds(NEL, NEL)], sem_g1)
        cpy0.wait()
        elem_stage(r0, n0, f0, 0, 0, sem_z0)
        cpy1.wait()
        elem_stage(r1, n1, f1, 1, 1, sem_z1)
        return c

    lax.fori_loop(0, RPW // 2, pair, 0)

    pltpu.make_async_copy(z_buf.at[pl.ds(0, D_SAE)],
                          z_hbm.at[row0 + RPW - 2], sem_z0).wait()
    pltpu.make_async_copy(z_buf.at[pl.ds(D_SAE, D_SAE)],
                          z_hbm.at[row0 + RPW - 1], sem_z1).wait()


def _sc_topk(pre, cmax):
    mesh = plsc.VectorSubcoreMesh(core_axis_name="c", subcore_axis_name="s",
                                  num_cores=NC, num_subcores=NS)
    kfn = pl.kernel(
        _sc_topk_kernel,
        out_type=[
            jax.ShapeDtypeStruct((B, D_SAE), jnp.float32),
            jax.ShapeDtypeStruct((B * KTOP,), jnp.float32),
            jax.ShapeDtypeStruct((B * KTOP,), jnp.int32),
        ],
        mesh=mesh,
        compiler_params=pltpu.CompilerParams(needs_layout_passes=False),
        scratch_types=[
            pltpu.VMEM((NCHUNK,), jnp.float32),
            pltpu.VMEM((NCHUNK,), jnp.int32),
            pltpu.VMEM((4096,), jnp.int32),
            pltpu.VMEM((NCHUNK + 16,), jnp.int32),
            pltpu.VMEM((2 * NEL,), jnp.int32),
            pltpu.VMEM((2 * NEL,), jnp.int32),
            pltpu.VMEM((144,), jnp.int32),
            pltpu.VMEM((144,), jnp.float32),
            pltpu.VMEM((144,), jnp.int32),
            pltpu.VMEM((2 * NEL,), jnp.float32),
            pltpu.VMEM((80,), jnp.float32),
            pltpu.VMEM((160,), jnp.int32),
            pltpu.VMEM((2 * D_SAE,), jnp.float32),
            pltpu.SemaphoreType.DMA,
            pltpu.SemaphoreType.DMA,
            pltpu.SemaphoreType.DMA,
            pltpu.SemaphoreType.DMA,
        ],
    )
    zf, vf, inf = kfn(pre.reshape(-1), cmax)
    return (zf, vf.reshape(B, KTOP), inf.reshape(B, KTOP))


KB = 1024
NKB = D_SAE // KB


def _decode_body(z_ref, w_ref, bd_ref, out_ref):
    kb = pl.program_id(0)

    @pl.when(kb == 0)
    def _():
        out_ref[...] = jnp.broadcast_to(bd_ref[...], (B, D_IN))

    out_ref[...] += lax.dot_general(
        z_ref[...].astype(jnp.bfloat16), w_ref[...].astype(jnp.bfloat16),
        (((1,), (1,)), ((), ())), preferred_element_type=jnp.float32)


def _decode(z, w_dec, b_dec):
    return pl.pallas_call(
        _decode_body,
        grid=(NKB,),
        in_specs=[
            pl.BlockSpec((B, KB), lambda k: (0, k)),
            pl.BlockSpec((D_IN, KB), lambda k: (0, k)),
            pl.BlockSpec((1, D_IN), lambda k: (0, 0)),
        ],
        out_specs=pl.BlockSpec((B, D_IN), lambda k: (0, 0)),
        out_shape=jax.ShapeDtypeStruct((B, D_IN), jnp.float32),
        compiler_params=pltpu.CompilerParams(
            dimension_semantics=("arbitrary",)),
    )(z, w_dec, b_dec.reshape(1, D_IN))


LRB = 256
NLRB = B // LRB


def _loss_body(xh_ref, x_ref, out_ref):
    rb = pl.program_id(0)
    d = xh_ref[...] - x_ref[...]
    s = jnp.sum(d * d).reshape(1, 1)

    @pl.when(rb == 0)
    def _():
        out_ref[...] = jnp.zeros((1, 1), jnp.float32)

    out_ref[...] += s

    @pl.when(rb == NLRB - 1)
    def _():
        out_ref[...] = out_ref[...] / B


def _loss(x_hat, x):
    return pl.pallas_call(
        _loss_body,
        grid=(NLRB,),
        in_specs=[
            pl.BlockSpec((LRB, D_IN), lambda r: (r, 0)),
            pl.BlockSpec((LRB, D_IN), lambda r: (r, 0)),
        ],
        out_specs=pl.BlockSpec((1, 1), lambda r: (0, 0)),
        out_shape=jax.ShapeDtypeStruct((1, 1), jnp.float32),
        compiler_params=pltpu.CompilerParams(
            dimension_semantics=("arbitrary",)),
    )(x_hat, x)


def kernel(x, W_enc, b_enc, W_dec, b_dec):
    pre, cmax = _encode(x, W_enc, b_enc, b_dec)
    z, _vals, _idx = _sc_topk(pre, cmax)
    x_hat = _decode(z, W_dec, b_dec)
    loss = _loss(x_hat, x)
    return (loss.reshape(()), x_hat, z)
